# baseline (speedup 1.0000x reference)
import functools

import jax
import jax.numpy as jnp
from jax import lax
from jax.experimental import pallas as pl
from jax.experimental.pallas import tpu as pltpu

_NUM_BINS = 10
_BLOCK_ROWS = 256
_BLOCK_COLS = 1024


def _mmce_kernel(probs_ref, tgt_ref, lower_ref, upper_ref, out_ref, acc_ref,
                 *, num_blocks, n_rows, n_cols):
    i = pl.program_id(0)

    @pl.when(i == 0)
    def _init():
        acc_ref[...] = jnp.zeros_like(acc_ref)

    t = tgt_ref[...]
    m = None
    k = None
    for j in range(_BLOCK_COLS // 128):
        xj = probs_ref[:, j * 128:(j + 1) * 128]
        colj = lax.broadcasted_iota(jnp.int32, xj.shape, 1) + j * 128
        if (j + 1) * 128 > n_cols:
            xj = jnp.where(colj < n_cols, xj, -1.0)
        kj = jnp.where(colj == t, xj, -1.0)
        m = xj if m is None else jnp.maximum(m, xj)
        k = kj if k is None else jnp.maximum(k, kj)
    conf = jnp.max(m, axis=1, keepdims=True)
    pv = jnp.max(k, axis=1, keepdims=True)
    acc = (pv == conf).astype(jnp.float32)

    lower = lower_ref[...]
    upper = upper_ref[...]
    in_bin = ((conf > lower) & (conf <= upper)).astype(jnp.float32)

    acc_ref[0:1, :] += jnp.sum(in_bin, axis=0, keepdims=True)
    acc_ref[1:2, :] += jnp.sum(in_bin * acc, axis=0, keepdims=True)
    acc_ref[2:3, :] += jnp.sum(in_bin * conf, axis=0, keepdims=True)

    @pl.when(i == num_blocks - 1)
    def _finalize():
        tcnt = acc_ref[0:1, :]
        tasum = acc_ref[1:2, :]
        tcsum = acc_ref[2:3, :]
        safe = jnp.maximum(tcnt, 1.0)
        bin_err = jnp.abs(tasum / safe - tcsum / safe)
        contrib = jnp.where(tcnt > 0, (tcnt / n_rows) * bin_err, 0.0)
        out_ref[...] = jnp.sum(contrib, axis=1, keepdims=True)


def kernel(probs, targets):
    n_rows, n_cols = probs.shape
    num_blocks = n_rows // _BLOCK_ROWS
    bounds = jnp.linspace(0.0, 1.0, _NUM_BINS + 1)
    lower = bounds[:_NUM_BINS].reshape(1, _NUM_BINS)
    upper = bounds[1:].reshape(1, _NUM_BINS)
    tgt2d = targets.reshape(n_rows, 1).astype(jnp.int32)

    out = pl.pallas_call(
        functools.partial(_mmce_kernel, num_blocks=num_blocks,
                          n_rows=n_rows, n_cols=n_cols),
        grid=(num_blocks,),
        in_specs=[
            pl.BlockSpec((_BLOCK_ROWS, _BLOCK_COLS), lambda i: (i, 0)),
            pl.BlockSpec((_BLOCK_ROWS, 1), lambda i: (i, 0)),
            pl.BlockSpec((1, _NUM_BINS), lambda i: (0, 0)),
            pl.BlockSpec((1, _NUM_BINS), lambda i: (0, 0)),
        ],
        out_specs=pl.BlockSpec((1, 1), lambda i: (0, 0)),
        out_shape=jax.ShapeDtypeStruct((1, 1), jnp.float32),
        scratch_shapes=[pltpu.VMEM((3, _NUM_BINS), jnp.float32)],
    )(probs, tgt2d, lower, upper)
    return out[0, 0]


# speedup vs baseline: 1.2612x; 1.2612x over previous
import functools

import jax
import jax.numpy as jnp
from jax import lax
from jax.experimental import pallas as pl
from jax.experimental.pallas import tpu as pltpu

_NUM_BINS = 10
_BLOCK_ROWS = 1024
_BLOCK_COLS = 1024


def _mmce_kernel(probs_ref, tgt_ref, lower_ref, upper_ref, out_ref, acc_ref,
                 *, num_blocks, n_rows, n_cols):
    i = pl.program_id(0)

    @pl.when(i == 0)
    def _init():
        acc_ref[...] = jnp.zeros_like(acc_ref)

    t = tgt_ref[...]
    m = None
    k = None
    for j in range(_BLOCK_COLS // 128):
        xj = probs_ref[:, j * 128:(j + 1) * 128]
        colj = lax.broadcasted_iota(jnp.int32, xj.shape, 1) + j * 128
        if (j + 1) * 128 > n_cols:
            xj = jnp.where(colj < n_cols, xj, -1.0)
        kj = jnp.where(colj == t, xj, -1.0)
        m = xj if m is None else jnp.maximum(m, xj)
        k = kj if k is None else jnp.maximum(k, kj)
    conf = jnp.max(m, axis=1, keepdims=True)
    pv = jnp.max(k, axis=1, keepdims=True)
    acc = (pv == conf).astype(jnp.float32)

    lower = lower_ref[...]
    upper = upper_ref[...]
    in_bin = ((conf > lower) & (conf <= upper)).astype(jnp.float32)

    acc_ref[0:1, :] += jnp.sum(in_bin, axis=0, keepdims=True)
    acc_ref[1:2, :] += jnp.sum(in_bin * acc, axis=0, keepdims=True)
    acc_ref[2:3, :] += jnp.sum(in_bin * conf, axis=0, keepdims=True)

    @pl.when(i == num_blocks - 1)
    def _finalize():
        tcnt = acc_ref[0:1, :]
        tasum = acc_ref[1:2, :]
        tcsum = acc_ref[2:3, :]
        safe = jnp.maximum(tcnt, 1.0)
        bin_err = jnp.abs(tasum / safe - tcsum / safe)
        contrib = jnp.where(tcnt > 0, (tcnt / n_rows) * bin_err, 0.0)
        out_ref[...] = jnp.sum(contrib, axis=1, keepdims=True)


def kernel(probs, targets):
    n_rows, n_cols = probs.shape
    num_blocks = n_rows // _BLOCK_ROWS
    bounds = jnp.linspace(0.0, 1.0, _NUM_BINS + 1)
    lower = bounds[:_NUM_BINS].reshape(1, _NUM_BINS)
    upper = bounds[1:].reshape(1, _NUM_BINS)
    tgt2d = targets.reshape(n_rows, 1).astype(jnp.int32)

    out = pl.pallas_call(
        functools.partial(_mmce_kernel, num_blocks=num_blocks,
                          n_rows=n_rows, n_cols=n_cols),
        grid=(num_blocks,),
        in_specs=[
            pl.BlockSpec((_BLOCK_ROWS, _BLOCK_COLS), lambda i: (i, 0)),
            pl.BlockSpec((_BLOCK_ROWS, 1), lambda i: (i, 0)),
            pl.BlockSpec((1, _NUM_BINS), lambda i: (0, 0)),
            pl.BlockSpec((1, _NUM_BINS), lambda i: (0, 0)),
        ],
        out_specs=pl.BlockSpec((1, 1), lambda i: (0, 0)),
        out_shape=jax.ShapeDtypeStruct((1, 1), jnp.float32),
        scratch_shapes=[pltpu.VMEM((3, _NUM_BINS), jnp.float32)],
    )(probs, tgt2d, lower, upper)
    return out[0, 0]
